# baseline (speedup 1.0000x reference)
import functools

import jax
import jax.numpy as jnp
from jax import lax
from jax.experimental import pallas as pl
from jax.experimental.pallas import tpu as pltpu
from jax.experimental.pallas import tpu_sc as plsc

_NUM_USERS = 5000
_NUM_ITEMS = 5000
_N_NODES = _NUM_USERS + _NUM_ITEMS
_EMB = 128
_N_EDGES = 320000

_NW = 32
_B = 128
_NCH = 10
_BPC = 8
_E_PAD = _NW * _NCH * _BPC * _B
_TILES = 16
_N_PAD = 10240
_ROWS_PER_TILE = _N_PAD // _TILES

_mesh = plsc.VectorSubcoreMesh(core_axis_name="c", subcore_axis_name="s")


@functools.partial(
    pl.kernel,
    out_type=jax.ShapeDtypeStruct((2, _N_PAD, _EMB), jnp.float32),
    mesh=_mesh,
    scratch_types=[
        pltpu.VMEM((_BPC, _B), jnp.int32),
        pltpu.VMEM((_BPC, _B), jnp.int32),
        pltpu.VMEM((_BPC, _B), jnp.float32),
        pltpu.VMEM((2 * _B, _EMB), jnp.float32),
        pltpu.VMEM_SHARED((_N_PAD, _EMB), jnp.float32),
        pltpu.SemaphoreType.DMA,
        pltpu.SemaphoreType.DMA,
    ],
)
def _spmm_layer(x_hbm, col_hbm, row_hbm, val_hbm, z_hbm, y_hbm,
                colv, rowv, valv, rows, acc, gsem, ssem):
    c = lax.axis_index("c")
    s = lax.axis_index("s")
    w = s * 2 + c

    slab = pl.ds(s * _ROWS_PER_TILE, _ROWS_PER_TILE)
    pltpu.sync_copy(z_hbm.at[slab], acc.at[slab])
    plsc.subcore_barrier()

    def scale(j, base):
        def blk(k, c2):
            e0 = k * 16
            vals16 = valv[j, pl.ds(e0, 16)]
            for jj in range(16):
                v = vals16[jj]
                for d in range(_EMB // 16):
                    sl = pl.ds(d * 16, 16)
                    rows[base + e0 + jj, sl] = rows[base + e0 + jj, sl] * v
            return c2
        lax.fori_loop(0, _B // 16, blk, 0)

    def chunk_body(ch, carry0):
        pltpu.sync_copy(col_hbm.at[w, ch], colv)
        pltpu.sync_copy(row_hbm.at[w, ch], rowv)
        pltpu.sync_copy(val_hbm.at[w, ch], valv)

        pltpu.async_copy(x_hbm.at[colv.at[0]], rows.at[pl.ds(0, _B)], gsem)

        def batch_body(i, carry):
            base = pl.multiple_of((i % 2) * _B, _B)
            nbase = pl.multiple_of(((i + 1) % 2) * _B, _B)
            cur = rows.at[pl.ds(base, _B)]
            pltpu.make_async_copy(x_hbm.at[colv.at[0]], cur, gsem).wait()

            @pl.when(i > 0)
            def _():
                pltpu.make_async_copy(
                    rows.at[pl.ds(nbase, _B)], acc.at[rowv.at[0]], ssem).wait()

            @pl.when(i < _BPC - 1)
            def _():
                pltpu.async_copy(
                    x_hbm.at[colv.at[i + 1]], rows.at[pl.ds(nbase, _B)], gsem)

            scale(i, base)
            pltpu.async_copy(cur, acc.at[rowv.at[i]], ssem, add=True)
            return carry
        lax.fori_loop(0, _BPC, batch_body, 0)
        lbase = pl.multiple_of(((_BPC - 1) % 2) * _B, _B)
        pltpu.make_async_copy(
            rows.at[pl.ds(lbase, _B)], acc.at[rowv.at[0]], ssem).wait()
        return carry0
    lax.fori_loop(0, _NCH, chunk_body, 0)

    plsc.subcore_barrier()
    pltpu.sync_copy(acc.at[slab], y_hbm.at[c, slab])


_TC_BLK = 1000


def _tc_call(body, n_in, n_out):
    return pl.pallas_call(
        body,
        grid=(_N_NODES // _TC_BLK,),
        in_specs=[pl.BlockSpec((_TC_BLK, _EMB), lambda i: (i, 0))] * n_in,
        out_specs=[pl.BlockSpec((_TC_BLK, _EMB), lambda i: (i, 0))] * n_out,
        out_shape=[jax.ShapeDtypeStruct((_N_NODES, _EMB), jnp.float32)] * n_out,
    )


def _add2_body(a, b, o):
    o[...] = a[...] + b[...]


def _comb2_body(a, b, s1, x_o, s_o):
    x = a[...] + b[...]
    x_o[...] = x
    s_o[...] = s1[...] + x


def _final_body(x0, s2, a, b, o):
    o[...] = (x0[...] + s2[...] + a[...] + b[...]) * 0.25


_add2 = _tc_call(_add2_body, 2, 1)
_comb2 = _tc_call(_comb2_body, 3, 2)
_final = _tc_call(_final_body, 4, 1)


def kernel(user_emb, item_emb, adj_index, adj_values):
    x0 = jnp.concatenate([user_emb, item_emb], axis=0)
    npad = _E_PAD - _N_EDGES
    pad_dst = _N_NODES + (jnp.arange(npad, dtype=jnp.int32) % (_N_PAD - _N_NODES))
    pad_src = jnp.arange(npad, dtype=jnp.int32) % _N_NODES
    row = jnp.concatenate([adj_index[0], pad_dst]).reshape(_NW, _NCH, _BPC, _B)
    col = jnp.concatenate([adj_index[1], pad_src]).reshape(_NW, _NCH, _BPC, _B)
    val = jnp.concatenate([adj_values, jnp.zeros((npad,), jnp.float32)])
    val = val.reshape(_NW, _NCH, _BPC, _B)
    z = jnp.zeros((_N_PAD, _EMB), jnp.float32)

    y1 = _spmm_layer(x0, col, row, val, z)
    (x1,) = _add2(y1[0, :_N_NODES], y1[1, :_N_NODES])
    y2 = _spmm_layer(x1, col, row, val, z)
    x2, s2 = _comb2(y2[0, :_N_NODES], y2[1, :_N_NODES], x1)
    y3 = _spmm_layer(x2, col, row, val, z)
    (fin,) = _final(x0, s2, y3[0, :_N_NODES], y3[1, :_N_NODES])

    return fin[:_NUM_USERS], fin[_NUM_USERS:]


# speedup vs baseline: 1.1540x; 1.1540x over previous
import functools

import jax
import jax.numpy as jnp
from jax import lax
from jax.experimental import pallas as pl
from jax.experimental.pallas import tpu as pltpu
from jax.experimental.pallas import tpu_sc as plsc

_NUM_USERS = 5000
_NUM_ITEMS = 5000
_N_NODES = _NUM_USERS + _NUM_ITEMS
_EMB = 128
_N_EDGES = 320000

_NW = 32
_B = 128
_NCH = 10
_BPC = 8
_E_PAD = _NW * _NCH * _BPC * _B
_TILES = 16
_N_PAD = 10240
_ROWS_PER_TILE = _N_PAD // _TILES

_mesh = plsc.VectorSubcoreMesh(core_axis_name="c", subcore_axis_name="s")


@functools.partial(
    pl.kernel,
    out_type=jax.ShapeDtypeStruct((2, _N_PAD, _EMB), jnp.float32),
    mesh=_mesh,
    scratch_types=[
        pltpu.VMEM((2 * 3 * _BPC, _B), jnp.int32),
        pltpu.VMEM((2 * _B, _EMB), jnp.float32),
        pltpu.VMEM_SHARED((_N_PAD, _EMB), jnp.float32),
        pltpu.SemaphoreType.DMA,
        pltpu.SemaphoreType.DMA,
    ],
)
def _spmm_layer(x_hbm, edges_hbm, z_hbm, y_hbm,
                edv, rows, acc, gsem, stgsem):
    c = lax.axis_index("c")
    s = lax.axis_index("s")
    w = s * 2 + c

    slab = pl.ds(s * _ROWS_PER_TILE, _ROWS_PER_TILE)
    pltpu.sync_copy(z_hbm.at[slab], acc.at[slab])
    plsc.subcore_barrier()

    def scale(vrow, base):
        def blk(k, c2):
            e0 = k * 16
            vals16 = jax.lax.bitcast_convert_type(edv[vrow, pl.ds(e0, 16)], jnp.float32)
            for jj in range(16):
                v = vals16[jj]
                for d in range(_EMB // 16):
                    sl = pl.ds(d * 16, 16)
                    rows[base + e0 + jj, sl] = rows[base + e0 + jj, sl] * v
            return c2
        lax.fori_loop(0, _B // 16, blk, 0)

    _NB = _NCH * _BPC

    _SEC = 3 * _BPC

    pltpu.sync_copy(edges_hbm.at[w, 0], edv.at[pl.ds(0, _SEC)])
    pltpu.async_copy(x_hbm.at[edv.at[0]], rows.at[pl.ds(0, _B)], gsem)

    def batch_body(i, carry):
        local = i % _BPC
        ch = i // _BPC
        par = ch % 2
        base = pl.multiple_of((i % 2) * _B, _B)
        nbase = pl.multiple_of(((i + 1) % 2) * _B, _B)
        cur = rows.at[pl.ds(base, _B)]

        pbase = par * _SEC
        @pl.when(jnp.logical_and(local == 0, ch + 1 < _NCH))
        def _():
            obase = pl.multiple_of((1 - par) * _SEC, 8)
            pltpu.async_copy(
                edges_hbm.at[w, ch + 1], edv.at[pl.ds(obase, _SEC)], stgsem)

        @pl.when(jnp.logical_and(local == _BPC - 1, i + 1 < _NB))
        def _():
            pltpu.make_async_copy(
                edges_hbm.at[w, 0], edv.at[pl.ds(0, _SEC)], stgsem).wait()

        pltpu.make_async_copy(x_hbm.at[edv.at[0]], cur, gsem).wait()

        @pl.when(i + 1 < _NB)
        def _():
            ni = i + 1
            pltpu.async_copy(
                x_hbm.at[edv.at[((ni // _BPC) % 2) * _SEC + ni % _BPC]],
                rows.at[pl.ds(nbase, _B)], gsem)

        scale(pbase + 2 * _BPC + local, base)
        pltpu.sync_copy(cur, acc.at[edv.at[pbase + _BPC + local]], add=True)
        return carry
    lax.fori_loop(0, _NB, batch_body, 0)

    plsc.subcore_barrier()
    pltpu.sync_copy(acc.at[slab], y_hbm.at[c, slab])


_TC_BLK = 1000


def _tc_call(body, n_in, n_out):
    return pl.pallas_call(
        body,
        grid=(_N_NODES // _TC_BLK,),
        in_specs=[pl.BlockSpec((_TC_BLK, _EMB), lambda i: (i, 0))] * n_in,
        out_specs=[pl.BlockSpec((_TC_BLK, _EMB), lambda i: (i, 0))] * n_out,
        out_shape=[jax.ShapeDtypeStruct((_N_NODES, _EMB), jnp.float32)] * n_out,
    )


def _add2_body(a, b, o):
    o[...] = a[...] + b[...]


def _comb2_body(a, b, s1, x_o, s_o):
    x = a[...] + b[...]
    x_o[...] = x
    s_o[...] = s1[...] + x


def _final_body(x0, s2, a, b, o):
    o[...] = (x0[...] + s2[...] + a[...] + b[...]) * 0.25


_add2 = _tc_call(_add2_body, 2, 1)
_comb2 = _tc_call(_comb2_body, 3, 2)
_final = _tc_call(_final_body, 4, 1)


def kernel(user_emb, item_emb, adj_index, adj_values):
    x0 = jnp.concatenate([user_emb, item_emb], axis=0)
    npad = _E_PAD - _N_EDGES
    pad_dst = _N_NODES + (jnp.arange(npad, dtype=jnp.int32) % (_N_PAD - _N_NODES))
    pad_src = jnp.arange(npad, dtype=jnp.int32) % _N_NODES
    row = jnp.concatenate([adj_index[0], pad_dst]).reshape(_NW, _NCH, _BPC, _B)
    col = jnp.concatenate([adj_index[1], pad_src]).reshape(_NW, _NCH, _BPC, _B)
    val = jnp.concatenate([adj_values, jnp.zeros((npad,), jnp.float32)])
    valbits = jax.lax.bitcast_convert_type(val, jnp.int32)
    valbits = valbits.reshape(_NW, _NCH, _BPC, _B)
    edges = jnp.stack([col, row, valbits], axis=2).reshape(_NW, _NCH, 3 * _BPC, _B)
    z = jnp.zeros((_N_PAD, _EMB), jnp.float32)

    y1 = _spmm_layer(x0, edges, z)
    (x1,) = _add2(y1[0, :_N_NODES], y1[1, :_N_NODES])
    y2 = _spmm_layer(x1, edges, z)
    x2, s2 = _comb2(y2[0, :_N_NODES], y2[1, :_N_NODES], x1)
    y3 = _spmm_layer(x2, edges, z)
    (fin,) = _final(x0, s2, y3[0, :_N_NODES], y3[1, :_N_NODES])

    return fin[:_NUM_USERS], fin[_NUM_USERS:]


# speedup vs baseline: 1.2200x; 1.0572x over previous
import functools

import jax
import jax.numpy as jnp
from jax import lax
from jax.experimental import pallas as pl
from jax.experimental.pallas import tpu as pltpu
from jax.experimental.pallas import tpu_sc as plsc

_NUM_USERS = 5000
_NUM_ITEMS = 5000
_N_NODES = _NUM_USERS + _NUM_ITEMS
_EMB = 128
_N_EDGES = 320000

_NW = 32
_B = 128
_NCH = 10
_BPC = 8
_E_PAD = _NW * _NCH * _BPC * _B
_TILES = 16
_N_PAD = 10240
_ROWS_PER_TILE = _N_PAD // _TILES

_mesh = plsc.VectorSubcoreMesh(core_axis_name="c", subcore_axis_name="s")


@functools.partial(
    pl.kernel,
    out_type=jax.ShapeDtypeStruct((2, _N_PAD, _EMB), jnp.float32),
    mesh=_mesh,
    scratch_types=[
        pltpu.VMEM((2 * 3 * _BPC, _B), jnp.int32),
        pltpu.VMEM((2 * _B, _EMB), jnp.float32),
        pltpu.VMEM_SHARED((_N_PAD, _EMB), jnp.float32),
        pltpu.SemaphoreType.DMA,
        pltpu.SemaphoreType.DMA,
    ],
)
def _spmm_layer(x_hbm, edges_hbm, z_hbm, y_hbm,
                edv, rows, acc, gsem, stgsem):
    c = lax.axis_index("c")
    s = lax.axis_index("s")
    w = s * 2 + c

    slab = pl.ds(s * _ROWS_PER_TILE, _ROWS_PER_TILE)
    pltpu.async_copy(z_hbm.at[slab], acc.at[slab], stgsem)

    def scale(vrow, base):
        def blk(k, c2):
            e0 = k * 16
            vals16 = jax.lax.bitcast_convert_type(edv[vrow, pl.ds(e0, 16)], jnp.float32)
            for jj in range(16):
                v = vals16[jj]
                for d in range(_EMB // 16):
                    sl = pl.ds(d * 16, 16)
                    rows[base + e0 + jj, sl] = rows[base + e0 + jj, sl] * v
            return c2
        lax.fori_loop(0, _B // 16, blk, 0)

    _NB = _NCH * _BPC

    _SEC = 3 * _BPC

    pltpu.sync_copy(edges_hbm.at[w, 0], edv.at[pl.ds(0, _SEC)])
    pltpu.async_copy(x_hbm.at[edv.at[0]], rows.at[pl.ds(0, _B)], gsem)
    pltpu.make_async_copy(z_hbm.at[slab], acc.at[slab], stgsem).wait()
    plsc.subcore_barrier()

    def batch_body(i, carry):
        local = i % _BPC
        ch = i // _BPC
        par = ch % 2
        base = pl.multiple_of((i % 2) * _B, _B)
        nbase = pl.multiple_of(((i + 1) % 2) * _B, _B)
        cur = rows.at[pl.ds(base, _B)]

        pbase = par * _SEC
        @pl.when(jnp.logical_and(local == 0, ch + 1 < _NCH))
        def _():
            obase = pl.multiple_of((1 - par) * _SEC, 8)
            pltpu.async_copy(
                edges_hbm.at[w, ch + 1], edv.at[pl.ds(obase, _SEC)], stgsem)

        @pl.when(jnp.logical_and(local == _BPC - 1, i + 1 < _NB))
        def _():
            pltpu.make_async_copy(
                edges_hbm.at[w, 0], edv.at[pl.ds(0, _SEC)], stgsem).wait()

        pltpu.make_async_copy(x_hbm.at[edv.at[0]], cur, gsem).wait()

        @pl.when(i + 1 < _NB)
        def _():
            ni = i + 1
            pltpu.async_copy(
                x_hbm.at[edv.at[((ni // _BPC) % 2) * _SEC + ni % _BPC]],
                rows.at[pl.ds(nbase, _B)], gsem)

        scale(pbase + 2 * _BPC + local, base)
        pltpu.sync_copy(cur, acc.at[edv.at[pbase + _BPC + local]], add=True)
        return carry
    lax.fori_loop(0, _NB, batch_body, 0)

    plsc.subcore_barrier()
    pltpu.sync_copy(acc.at[slab], y_hbm.at[c, slab])


_TC_BLK = 1000
_G = _N_NODES // _TC_BLK


def _half_spec(h):
    return pl.BlockSpec((1, _TC_BLK, _EMB), lambda i, _h=h: (_h, i, 0))


_FLAT_SPEC = pl.BlockSpec((_TC_BLK, _EMB), lambda i: (i, 0))
_OUT_SHAPE = jax.ShapeDtypeStruct((_N_NODES, _EMB), jnp.float32)


def _add2_body(a, b, o):
    o[...] = a[0] + b[0]


_add2 = pl.pallas_call(
    _add2_body, grid=(_G,),
    in_specs=[_half_spec(0), _half_spec(1)],
    out_specs=[_FLAT_SPEC], out_shape=[_OUT_SHAPE])


def _comb2_body(a, b, s1, x_o, s_o):
    x = a[0] + b[0]
    x_o[...] = x
    s_o[...] = s1[...] + x


_comb2 = pl.pallas_call(
    _comb2_body, grid=(_G,),
    in_specs=[_half_spec(0), _half_spec(1), _FLAT_SPEC],
    out_specs=[_FLAT_SPEC] * 2, out_shape=[_OUT_SHAPE] * 2)


def _final_body(x0, s2, a, b, o):
    o[...] = (x0[...] + s2[...] + a[0] + b[0]) * 0.25


_final = pl.pallas_call(
    _final_body, grid=(_G,),
    in_specs=[_FLAT_SPEC, _FLAT_SPEC, _half_spec(0), _half_spec(1)],
    out_specs=[_FLAT_SPEC], out_shape=[_OUT_SHAPE])


def kernel(user_emb, item_emb, adj_index, adj_values):
    x0 = jnp.concatenate([user_emb, item_emb], axis=0)
    npad = _E_PAD - _N_EDGES
    pad_dst = _N_NODES + (jnp.arange(npad, dtype=jnp.int32) % (_N_PAD - _N_NODES))
    pad_src = jnp.arange(npad, dtype=jnp.int32) % _N_NODES
    row = jnp.concatenate([adj_index[0], pad_dst]).reshape(_NW, _NCH, _BPC, _B)
    col = jnp.concatenate([adj_index[1], pad_src]).reshape(_NW, _NCH, _BPC, _B)
    val = jnp.concatenate([adj_values, jnp.zeros((npad,), jnp.float32)])
    valbits = jax.lax.bitcast_convert_type(val, jnp.int32)
    valbits = valbits.reshape(_NW, _NCH, _BPC, _B)
    edges = jnp.stack([col, row, valbits], axis=2).reshape(_NW, _NCH, 3 * _BPC, _B)
    z = jnp.zeros((_N_PAD, _EMB), jnp.float32)

    y1 = _spmm_layer(x0, edges, z)
    (x1,) = _add2(y1, y1)
    y2 = _spmm_layer(x1, edges, z)
    x2, s2 = _comb2(y2, y2, x1)
    y3 = _spmm_layer(x2, edges, z)
    (fin,) = _final(x0, s2, y3, y3)

    return fin[:_NUM_USERS], fin[_NUM_USERS:]
